# Initial kernel scaffold; baseline (speedup 1.0000x reference)
import jax
import jax.numpy as jnp
from jax.experimental import pallas as pl


def kernel(x, edge_index, W0, b0, g0, be0, W1, b1, g1, be1, W2, b2, g2, be2):
    raise NotImplementedError("write your pallas kernel here")


# speedup vs baseline: 3.2915x; 3.2915x over previous
import functools

import jax
import jax.numpy as jnp
from jax import lax
from jax.experimental import pallas as pl
from jax.experimental.pallas import tpu as pltpu
from jax.experimental.pallas import tpu_sc as plsc

N = 10000
E = 320000
D = 128
NC = 2
NS = 16
NW = NC * NS
C = 128
NCH = 80
ECAP = C * NCH
EPAD = NW * ECAP
SPLIT = 5000
HROWS = 5120
HZR = HROWS // NS
DUMMY_ROW = HROWS - 1
ACC_ROWS = 10240
ZR = ACC_ROWS // NS
DUMMY = N
DDEG = 16
R = 1000


@functools.lru_cache(maxsize=None)
def _mesh():
    return plsc.VectorSubcoreMesh(core_axis_name="c", subcore_axis_name="s",
                                  num_cores=NC, num_subcores=NS)


@functools.lru_cache(maxsize=None)
def _sc_bucketize_call():
    return pl.kernel(
        _sc_bucketize_body,
        out_type=[
            jax.ShapeDtypeStruct((2, NW, ECAP), jnp.int32),
            jax.ShapeDtypeStruct((2, NW, ECAP), jnp.int32),
            jax.ShapeDtypeStruct((2, NW, C), jnp.int32),
        ],
        mesh=_mesh(),
        compiler_params=pltpu.CompilerParams(needs_layout_passes=False),
        scratch_types=[
            pltpu.VMEM((NCH, C), jnp.int32),
            pltpu.VMEM((NCH, C), jnp.int32),
            pltpu.VMEM((ECAP + 16,), jnp.int32),
            pltpu.VMEM((ECAP + 16,), jnp.int32),
            pltpu.VMEM((ECAP + 16,), jnp.int32),
            pltpu.VMEM((ECAP + 16,), jnp.int32),
            pltpu.VMEM((C,), jnp.int32),
        ],
    )


def _sc_bucketize_body(src_hbm, dst_hbm, srcb_hbm, rowb_hbm, nch_hbm,
                       sin, din, slo, rlo, shi, rhi, ncv):
    c = lax.axis_index("c")
    s = lax.axis_index("s")
    wid = s * NC + c

    pltpu.sync_copy(src_hbm.at[wid], sin)
    pltpu.sync_copy(dst_hbm.at[wid], din)

    zero16 = jnp.zeros((16,), jnp.int32)
    dum16 = jnp.full((16,), DUMMY_ROW, jnp.int32)

    def _fill(i, _):
        slo[pl.ds(i * 16, 16)] = zero16
        rlo[pl.ds(i * 16, 16)] = dum16
        shi[pl.ds(i * 16, 16)] = zero16
        rhi[pl.ds(i * 16, 16)] = dum16
        return 0

    lax.fori_loop(0, (ECAP + 16) // 16, _fill, 0)

    def _comp(g, carry):
        olo, ohi = carry
        row = g // (C // 16)
        lane = (g % (C // 16)) * 16
        s16 = sin[row, pl.ds(lane, 16)]
        d16 = din[row, pl.ds(lane, 16)]
        mlo = d16 < SPLIT
        plsc.store_compressed(slo.at[pl.ds(olo, 16)], s16, mask=mlo)
        plsc.store_compressed(rlo.at[pl.ds(olo, 16)], d16, mask=mlo)
        mhi = jnp.logical_not(mlo)
        plsc.store_compressed(shi.at[pl.ds(ohi, 16)], s16, mask=mhi)
        plsc.store_compressed(rhi.at[pl.ds(ohi, 16)],
                              jnp.minimum(d16 - SPLIT, DUMMY_ROW), mask=mhi)
        nlo = plsc.all_reduce_population_count(mlo)[0]
        nhi = plsc.all_reduce_population_count(mhi)[0]
        return olo + nlo, ohi + nhi

    olo, ohi = lax.fori_loop(0, NCH * (C // 16), _comp, (0, 0))

    for h, (sb, rb, cnt) in enumerate(((slo, rlo, olo), (shi, rhi, ohi))):
        nchunks = (cnt + C - 1) // C
        v = jnp.full((16,), nchunks, jnp.int32)

        def _nb(i, _, v=v, ncv=ncv):
            ncv[pl.ds(i * 16, 16)] = v
            return 0

        lax.fori_loop(0, C // 16, _nb, 0)
        pltpu.sync_copy(ncv, nch_hbm.at[h, wid])
        pltpu.sync_copy(sb.at[pl.ds(0, ECAP)], srcb_hbm.at[h, wid])
        pltpu.sync_copy(rb.at[pl.ds(0, ECAP)], rowb_hbm.at[h, wid])


@functools.lru_cache(maxsize=None)
def _sc_segsum_call():
    return pl.kernel(
        _sc_segsum_body,
        out_type=jax.ShapeDtypeStruct((2, NC, HROWS, D), jnp.float32),
        mesh=_mesh(),
        scratch_types=[
            pltpu.VMEM((NCH, C), jnp.int32),
            pltpu.VMEM((NCH, C), jnp.int32),
            pltpu.VMEM((C,), jnp.int32),
            pltpu.VMEM((C, D), jnp.float32),
            pltpu.VMEM((C, D), jnp.float32),
            pltpu.VMEM_SHARED((HROWS, D), jnp.float32),
            pltpu.SemaphoreType.DMA,
            pltpu.SemaphoreType.DMA,
        ],
    )


def _sc_segsum_body(h_hbm, srcb_hbm, rowb_hbm, nch_hbm, out_hbm,
                    src_v, row_v, ncv, buf, zbuf, acc, sem_g, sem_s):
    c = lax.axis_index("c")
    s = lax.axis_index("s")
    wid = s * NC + c

    zero16 = jnp.zeros((16,), jnp.float32)

    def _zb(i, _):
        zbuf[i // 8, pl.ds((i % 8) * 16, 16)] = zero16
        return 0

    lax.fori_loop(0, C * 8, _zb, 0)

    for h in range(2):
        for off, rows in ((0, C), (C, C), (2 * C, HZR - 2 * C)):
            pltpu.sync_copy(zbuf.at[pl.ds(0, rows)],
                            acc.at[pl.ds(s * HZR + off, rows)])

        pltpu.sync_copy(srcb_hbm.at[h, wid], src_v)
        pltpu.sync_copy(rowb_hbm.at[h, wid], row_v)
        pltpu.sync_copy(nch_hbm.at[h, wid], ncv)
        n = ncv[pl.ds(0, 16)][0]
        plsc.subcore_barrier()

        def _body(j, _):
            pltpu.async_copy(h_hbm.at[src_v.at[j]], buf, sem_g).wait()
            pltpu.async_copy(buf, acc.at[row_v.at[j]], sem_s, add=True).wait()
            return 0

        lax.fori_loop(0, n, _body, 0)
        plsc.subcore_barrier()

        pltpu.sync_copy(acc.at[pl.ds(s * HZR, HZR)],
                        out_hbm.at[h, c, pl.ds(s * HZR, HZR)])
        plsc.subcore_barrier()


def _tc_layer_body(acc_ref, hd_ref, dinv_ref, res_ref, x_ref, w_ref, b_ref,
                   g_ref, be_ref, f_ref, p_ref, h_ref, hdn_ref, dinvn_ref):
    a = acc_ref[0, 0] + acc_ref[0, 1]
    dinv_new = lax.rsqrt(a + 1.0)
    dinv = dinv_ref[...]
    z = dinv * (a + hd_ref[...])
    y = lax.dot_general(z, w_ref[...], (((1,), (1,)), ((), ())),
                        preferred_element_type=jnp.float32) + b_ref[...]
    mu = jnp.mean(y, axis=1, keepdims=True)
    var = jnp.mean((y - mu) ** 2, axis=1, keepdims=True)
    t = (y - mu) * lax.rsqrt(var + 1e-5) * g_ref[...] + be_ref[...]
    f = f_ref[...]
    hl = f * (jnp.maximum(t, 0.0) + res_ref[...]) + (1.0 - f) * t
    p = p_ref[...]
    h = p * x_ref[...] + (1.0 - p) * hl
    dn = p * dinv_new + (1.0 - p) * dinv
    h_ref[...] = h
    hdn_ref[...] = h * dn
    dinvn_ref[...] = dn


def _tc_layer(acc, hd, dinv, res, x, w, b, g, be, f, p):
    return pl.pallas_call(
        _tc_layer_body,
        grid=(N // R,),
        in_specs=[
            pl.BlockSpec((1, NC, R, D), lambda i: (i // 5, 0, i % 5, 0)),
            pl.BlockSpec((R, D), lambda i: (i, 0)),
            pl.BlockSpec((R, D), lambda i: (i, 0)),
            pl.BlockSpec((R, D), lambda i: (i, 0)),
            pl.BlockSpec((R, D), lambda i: (i, 0)),
            pl.BlockSpec((D, D), lambda i: (0, 0)),
            pl.BlockSpec((1, D), lambda i: (0, 0)),
            pl.BlockSpec((1, D), lambda i: (0, 0)),
            pl.BlockSpec((1, D), lambda i: (0, 0)),
            pl.BlockSpec((1, D), lambda i: (0, 0)),
            pl.BlockSpec((1, D), lambda i: (0, 0)),
        ],
        out_specs=[
            pl.BlockSpec((R, D), lambda i: (i, 0)),
            pl.BlockSpec((R, D), lambda i: (i, 0)),
            pl.BlockSpec((R, D), lambda i: (i, 0)),
        ],
        out_shape=[
            jax.ShapeDtypeStruct((N, D), jnp.float32),
            jax.ShapeDtypeStruct((N, D), jnp.float32),
            jax.ShapeDtypeStruct((N, D), jnp.float32),
        ],
    )(acc, hd, dinv, res, x, w, b, g, be, f, p)


def kernel(x, edge_index, W0, b0, g0, be0, W1, b1, g1, be1, W2, b2, g2, be2):
    src = edge_index[0].astype(jnp.int32)
    dst = edge_index[1].astype(jnp.int32)
    pad = EPAD - E
    src = jnp.concatenate([src, jnp.zeros((pad,), jnp.int32)])
    dst = jnp.concatenate([dst, jnp.full((pad,), DUMMY, jnp.int32)])
    src3 = src.reshape(NW, NCH, C)
    dst3 = dst.reshape(NW, NCH, C)

    srcb, rowb, nch = _sc_bucketize_call()(src3, dst3)
    srcb = srcb.reshape(2, NW, NCH, C)
    rowb = rowb.reshape(2, NW, NCH, C)

    x = x.astype(jnp.float32)
    ws = jnp.stack([W0, W0, W1, W2]).astype(jnp.float32)
    bs = jnp.stack([b0, b0, b1, b2]).reshape(4, 1, D).astype(jnp.float32)
    gs = jnp.stack([g0, g0, g1, g2]).reshape(4, 1, D).astype(jnp.float32)
    bes = jnp.stack([be0, be0, be1, be2]).reshape(4, 1, D).astype(jnp.float32)
    fs = jnp.broadcast_to(
        jnp.array([0.0, 1.0, 1.0, 0.0], jnp.float32)[:, None, None], (4, 1, D))
    ps = jnp.broadcast_to(
        jnp.array([1.0, 0.0, 0.0, 0.0], jnp.float32)[:, None, None], (4, 1, D))

    def _step(carry, xs):
        hd_c, res_c, dinv_c = carry
        w, b, g, be, f, p = xs
        acc = _sc_segsum_call()(hd_c, srcb, rowb, nch)
        h, hdn, dinvn = _tc_layer(acc, hd_c, dinv_c, res_c, x, w, b, g, be, f, p)
        return (hdn, h, dinvn), ()

    ones = jnp.ones((N, D), jnp.float32)
    (_, h, _), _ = lax.scan(_step, (ones, x, ones), (ws, bs, gs, bes, fs, ps))
    return h
